# baseline (speedup 1.0000x reference)
import functools

import jax
import jax.numpy as jnp
from jax import lax
from jax.experimental import pallas as pl
from jax.experimental.pallas import tpu as pltpu
from jax.experimental.pallas import tpu_sc as plsc

N = 10000
E = 160000
G = 128
EPS = 1e-5
NCLS = 10

QF = 128
NQ = 4
F = QF * NQ
R = 1000
NT = N // R

NCORES = 2
NSUB = 16
CH = 128
NCHT = E // CH
SCR = 10
SR = NCHT // SCR
TMAX = (SR + NSUB - 1) // NSUB
CPWF = (NCHT + NSUB - 1) // NSUB
NEXTRA = NCHT - (CPWF - 1) * NSUB
_DIAG_NO_SCATTER = False
_DIAG_NO_GATHER = False
NBUF = 3
IDXB = 5
RCH = 80
NRCH = N // RCH
RITER = (NRCH + NSUB - 1) // NSUB


def _deg_body(dst3_hbm, deg_hbm, idx_v, ones_v, row_v, acc_sh):
    c = lax.axis_index("c")
    s = lax.axis_index("s")

    @pl.when(c == 0)
    def _():
        one16 = jnp.full((16,), 1.0, jnp.float32)
        for i in range(CH // 16):
            ones_v[pl.ds(i * 16, 16)] = one16
        for i in range(RCH // 16):
            row_v[pl.ds(i * 16, 16)] = one16

        def init_body(k, _):
            ch = s + k * NSUB

            @pl.when(ch < NRCH)
            def _():
                pltpu.sync_copy(row_v, acc_sh.at[pl.ds(ch * RCH, RCH)])
            return _

        lax.fori_loop(0, RITER, init_body, None)
        plsc.subcore_barrier()

        for t in range(TMAX):
            sup = s + t * NSUB

            @pl.when(sup < SR)
            def _():
                pltpu.sync_copy(dst3_hbm.at[pl.ds(sup, 1), :, :], idx_v)
                for i in range(SCR):
                    pltpu.sync_copy(ones_v, acc_sh.at[idx_v.at[0, i]],
                                    add=True)
        plsc.subcore_barrier()

        def out_body(k, _):
            ch = s + k * NSUB

            @pl.when(ch < NRCH)
            def _():
                pltpu.sync_copy(acc_sh.at[pl.ds(ch * RCH, RCH)], row_v)
                pltpu.sync_copy(row_v, deg_hbm.at[pl.ds(ch * RCH, RCH)])
            return _

        lax.fori_loop(0, RITER, out_body, None)


def _agg_body(y_hbm, src_hbm, dst_hbm, agg_hbm,
              isrc_b, idst_b, stage_v,
              acc_sh, semg, sems, semi):
    c = lax.axis_index("c")
    s = lax.axis_index("s")
    nch = jnp.where(s < NEXTRA, CPWF, CPWF - 1)

    nrit = jnp.where(s < NRCH - (RITER - 1) * NSUB, RITER, RITER - 1)

    def _ring_rows(ext_ref, riter, to_acc):
        def slot(b):
            return stage_v.at[b, pl.ds(0, RCH), :]

        def rows(r, ref):
            ch = s + r * NSUB
            return ref.at[pl.ds(ch * RCH, RCH), :]

        def d_in(r, b):
            if to_acc:
                return pltpu.make_async_copy(rows(r, ext_ref), slot(b),
                                             semg.at[b])
            return pltpu.make_async_copy(rows(r, acc_sh), slot(b),
                                         semg.at[b])

        def d_out(r, b):
            if to_acc:
                return pltpu.make_async_copy(slot(b), rows(r, acc_sh),
                                             sems.at[b])
            return pltpu.make_async_copy(slot(b), rows(r, ext_ref),
                                         sems.at[b])

        def body(r, _):
            b = lax.rem(r, NBUF)

            @pl.when(r >= NBUF)
            def _():
                d_out(r - NBUF, b).wait()

            d_in(r, b).start()

            @pl.when(r >= 1)
            def _():
                bp = lax.rem(r - 1, NBUF)
                d_in(r - 1, bp).wait()
                d_out(r - 1, bp).start()
            return _

        lax.fori_loop(0, riter, body, None)
        rl = riter - 1
        bl = lax.rem(rl, NBUF)
        d_in(rl, bl).wait()
        d_out(rl, bl).start()
        for d in range(NBUF):
            rr = rl - d

            @pl.when(rr >= 0)
            def _():
                d_out(rr, lax.rem(rr, NBUF)).wait()

    for qi in range(NQ // NCORES):
        q = c * (NQ // NCORES) + qi
        yq = y_hbm.at[q]

        _ring_rows(yq, nrit, to_acc=True)
        plsc.subcore_barrier()

        def i_descs(k, b):
            off = (s + k * NSUB) * CH
            return (pltpu.make_async_copy(src_hbm.at[pl.ds(off, CH)],
                                          isrc_b.at[b], semi.at[b]),
                    pltpu.make_async_copy(dst_hbm.at[pl.ds(off, CH)],
                                          idst_b.at[b], semi.at[b]))

        def i_start(k, b):
            d0, d1 = i_descs(k, b)
            d0.start()
            d1.start()

        def i_wait(k, b):
            d0, d1 = i_descs(k, b)
            d0.wait()
            d1.wait()

        def g_desc(k, b):
            return pltpu.make_async_copy(
                yq.at[isrc_b.at[lax.rem(k, IDXB)]], stage_v.at[b],
                semg.at[b])

        def s_dst(k):
            return acc_sh.at[idst_b.at[lax.rem(k, IDXB)]]

        def s_wait(k, b):
            pltpu.make_async_copy(stage_v.at[b], s_dst(k), sems.at[b]).wait()

        def s_start(k, b):
            pltpu.async_copy(stage_v.at[b], s_dst(k), sems.at[b], add=True)

        i_start(0, 0)
        i_start(1, 1)

        def edge_body(k, _):
            b = lax.rem(k, NBUF)

            if not _DIAG_NO_SCATTER:
                @pl.when(k >= NBUF)
                def _():
                    s_wait(k - NBUF, b)

            @pl.when(k + 2 < nch)
            def _():
                i_start(k + 2, lax.rem(k + 2, IDXB))

            i_wait(k, lax.rem(k, IDXB))
            if not _DIAG_NO_GATHER:
                g_desc(k, b).start()

            @pl.when(k >= 1)
            def _():
                bp = lax.rem(k - 1, NBUF)
                if not _DIAG_NO_GATHER:
                    g_desc(k - 1, bp).wait()
                if not _DIAG_NO_SCATTER:
                    s_start(k - 1, bp)
            return _

        lax.fori_loop(0, nch, edge_body, None)
        kl = nch - 1
        bl = lax.rem(kl, NBUF)
        if not _DIAG_NO_GATHER:
            g_desc(kl, bl).wait()
        if not _DIAG_NO_SCATTER:
            s_start(kl, bl)
            for d in range(NBUF):
                kk = kl - d
                s_wait(kk, lax.rem(kk, NBUF))

        plsc.subcore_barrier()

        _ring_rows(agg_hbm.at[q], nrit, to_acc=False)


@functools.lru_cache(maxsize=None)
def _sc_kernels():
    mesh = plsc.VectorSubcoreMesh(
        core_axis_name="c", subcore_axis_name="s",
        num_cores=NCORES, num_subcores=NSUB)
    deg = pl.kernel(
        _deg_body,
        out_type=jax.ShapeDtypeStruct((N,), jnp.float32),
        mesh=mesh,
        scratch_types=[
            pltpu.VMEM((1, SCR, CH), jnp.int32),
            pltpu.VMEM((CH,), jnp.float32),
            pltpu.VMEM((RCH,), jnp.float32),
            pltpu.VMEM_SHARED((N,), jnp.float32),
        ],
    )
    agg = pl.kernel(
        _agg_body,
        out_type=jax.ShapeDtypeStruct((NQ, N, QF), jnp.float32),
        mesh=mesh,
        compiler_params=pltpu.CompilerParams(use_tc_tiling_on_sc=False),
        scratch_types=[
            pltpu.VMEM((IDXB, CH), jnp.int32),
            pltpu.VMEM((IDXB, CH), jnp.int32),
            pltpu.VMEM((NBUF, CH, QF), jnp.float32),
            pltpu.VMEM_SHARED((N, QF), jnp.float32),
            pltpu.SemaphoreType.DMA((NBUF,)),
            pltpu.SemaphoreType.DMA((NBUF,)),
            pltpu.SemaphoreType.DMA((IDXB,)),
        ],
    )
    return deg, agg


def _tc0_body(h_ref, w_ref, deg_ref, y_ref, dinv_ref):
    dinv = lax.rsqrt(deg_ref[...])
    dinv_ref[...] = dinv
    xw = jnp.dot(h_ref[...], w_ref[...], preferred_element_type=jnp.float32)
    y = xw * dinv
    for q in range(NQ):
        y_ref[q] = y[:, q * QF:(q + 1) * QF]


def _tc0(h, w0, deg):
    return pl.pallas_call(
        _tc0_body,
        grid=(NT,),
        in_specs=[
            pl.BlockSpec((R, 256), lambda j: (j, 0)),
            pl.BlockSpec((256, F), lambda j: (0, 0)),
            pl.BlockSpec((R, 1), lambda j: (j, 0)),
        ],
        out_specs=[
            pl.BlockSpec((NQ, R, QF), lambda j: (0, j, 0)),
            pl.BlockSpec((R, 1), lambda j: (j, 0)),
        ],
        out_shape=[
            jax.ShapeDtypeStruct((NQ, N, QF), jnp.float32),
            jax.ShapeDtypeStruct((N, 1), jnp.float32),
        ],
    )(h, w0, deg)


def _accum_stats(agg_ref, dinv, s_acc, ss_acc, j):
    @pl.when(j == 0)
    def _():
        s_acc[...] = jnp.zeros((NQ, QF), jnp.float32)
        ss_acc[...] = jnp.zeros((NQ, QF), jnp.float32)

    for q in range(NQ):
        t = agg_ref[q] * dinv
        s_acc[q, :] += jnp.sum(t, axis=0)
        ss_acc[q, :] += jnp.sum(t * t, axis=0)


def _bn_tile(agg_ref, dinv, gb, s_acc, ss_acc):
    mean = s_acc[...] * (1.0 / N)
    var = ss_acc[...] * (1.0 / N) - mean * mean
    rstd = lax.rsqrt(var + EPS)
    cols = []
    for q in range(NQ):
        t = agg_ref[q] * dinv
        cols.append((t - mean[q]) * rstd[q] * gb[q] + gb[NQ + q])
    return jnp.maximum(jnp.concatenate(cols, axis=1), 0.0)


def _make_tcmid(residual):
    def body(agg_ref, dinv_ref, gb_ref, *rest):
        if residual:
            xp_ref, w_ref, x_ref, y_ref, s_acc, ss_acc = rest
        else:
            w_ref, x_ref, y_ref, s_acc, ss_acc = rest
        p = pl.program_id(0)
        j = pl.program_id(1)
        dinv = dinv_ref[...]

        @pl.when(p == 0)
        def _():
            _accum_stats(agg_ref, dinv, s_acc, ss_acc, j)

        @pl.when(p == 1)
        def _():
            x = _bn_tile(agg_ref, dinv, gb_ref[...], s_acc, ss_acc)
            if residual:
                x = x + xp_ref[...]
            x_ref[...] = x
            y = jnp.dot(x, w_ref[...],
                        preferred_element_type=jnp.float32) * dinv
            for q in range(NQ):
                y_ref[q] = y[:, q * QF:(q + 1) * QF]

    in_specs = [
        pl.BlockSpec((NQ, R, QF), lambda p, j: (0, j, 0)),
        pl.BlockSpec((R, 1), lambda p, j: (j, 0)),
        pl.BlockSpec((2 * NQ, QF), lambda p, j: (0, 0)),
    ]
    if residual:
        in_specs.append(
            pl.BlockSpec((R, F), lambda p, j: (jnp.where(p == 1, j, 0), 0)))
    in_specs.append(pl.BlockSpec((F, F), lambda p, j: (0, 0)))

    def run(*args):
        return pl.pallas_call(
            body,
            grid=(2, NT),
            in_specs=in_specs,
            out_specs=[
                pl.BlockSpec((R, F), lambda p, j: (jnp.where(p == 1, j, 0), 0)),
                pl.BlockSpec((NQ, R, QF),
                             lambda p, j: (0, jnp.where(p == 1, j, 0), 0)),
            ],
            out_shape=[
                jax.ShapeDtypeStruct((N, F), jnp.float32),
                jax.ShapeDtypeStruct((NQ, N, QF), jnp.float32),
            ],
            scratch_shapes=[
                pltpu.VMEM((NQ, QF), jnp.float32),
                pltpu.VMEM((NQ, QF), jnp.float32),
            ],
        )(*args)

    return run


_tcmid_nores = _make_tcmid(False)
_tcmid_res = _make_tcmid(True)


def _tcfinal_body(agg_ref, dinv_ref, gb_ref, xp_ref, batch_ref,
                  wm_ref, bm_ref, out_ref, s_acc, ss_acc, sums_acc, cnt_acc):
    p = pl.program_id(0)
    j = pl.program_id(1)
    dinv = dinv_ref[...]

    @pl.when(p == 0)
    def _():
        _accum_stats(agg_ref, dinv, s_acc, ss_acc, j)

    @pl.when(p == 1)
    def _():
        @pl.when(j == 0)
        def _():
            sums_acc[...] = jnp.zeros((G, F), jnp.float32)
            cnt_acc[...] = jnp.zeros((G, 1), jnp.float32)

        x = _bn_tile(agg_ref, dinv, gb_ref[...], s_acc, ss_acc)
        x = x + xp_ref[...]
        b = batch_ref[...].reshape(1, R)
        gids = lax.broadcasted_iota(jnp.int32, (G, R), 0)
        ind = (gids == b).astype(jnp.float32)
        sums_acc[...] += jnp.dot(ind, x, preferred_element_type=jnp.float32)
        cnt_acc[...] += jnp.sum(ind, axis=1, keepdims=True)

        @pl.when(j == NT - 1)
        def _():
            hg = sums_acc[...] / jnp.maximum(cnt_acc[...], 1.0)
            out_ref[...] = (
                jnp.dot(hg, wm_ref[...], preferred_element_type=jnp.float32)
                + bm_ref[0:1, :])


def _tcfinal(agg, dinv, gb, xp, batch3, wm, bm8):
    return pl.pallas_call(
        _tcfinal_body,
        grid=(2, NT),
        in_specs=[
            pl.BlockSpec((NQ, R, QF), lambda p, j: (0, j, 0)),
            pl.BlockSpec((R, 1), lambda p, j: (j, 0)),
            pl.BlockSpec((2 * NQ, QF), lambda p, j: (0, 0)),
            pl.BlockSpec((R, F), lambda p, j: (jnp.where(p == 1, j, 0), 0)),
            pl.BlockSpec((1, 1, R),
                         lambda p, j: (jnp.where(p == 1, j, 0), 0, 0)),
            pl.BlockSpec((F, NCLS), lambda p, j: (0, 0)),
            pl.BlockSpec((8, NCLS), lambda p, j: (0, 0)),
        ],
        out_specs=pl.BlockSpec((G, NCLS), lambda p, j: (0, 0)),
        out_shape=jax.ShapeDtypeStruct((G, NCLS), jnp.float32),
        scratch_shapes=[
            pltpu.VMEM((NQ, QF), jnp.float32),
            pltpu.VMEM((NQ, QF), jnp.float32),
            pltpu.VMEM((G, F), jnp.float32),
            pltpu.VMEM((G, 1), jnp.float32),
        ],
    )(agg, dinv, gb, xp, batch3, wm, bm8)


def kernel(h, edge_index, batch, e, W0, b0, g0, be0, W1, b1, g1, be1,
           W2, b2, g2, be2, W3, b3, g3, be3, Wm, bm):
    src = edge_index[0]
    dst = edge_index[1]
    dst3 = dst.reshape(SR, SCR, CH)
    _deg_kernel, _agg_kernel = _sc_kernels()

    deg = _deg_kernel(dst3).reshape(N, 1)
    y, dinv = _tc0(h, W0, deg)

    gbs = [jnp.concatenate([g.reshape(NQ, QF), be.reshape(NQ, QF)], axis=0)
           for g, be in ((g0, be0), (g1, be1), (g2, be2), (g3, be3))]
    ws = [W1, W2, W3]

    x = None
    for i in range(3):
        agg = _agg_kernel(y, src, dst)
        if i == 0:
            x, y = _tcmid_nores(agg, dinv, gbs[i], ws[i])
        else:
            x, y = _tcmid_res(agg, dinv, gbs[i], x, ws[i])

    agg = _agg_kernel(y, src, dst)
    batch3 = batch.reshape(NT, 1, R)
    bm8 = jnp.broadcast_to(bm.reshape(1, NCLS), (8, NCLS))
    return _tcfinal(agg, dinv, gbs[3], x, batch3, Wm, bm8)


# speedup vs baseline: 1.0000x; 1.0000x over previous
import functools

import jax
import jax.numpy as jnp
from jax import lax
from jax.experimental import pallas as pl
from jax.experimental.pallas import tpu as pltpu
from jax.experimental.pallas import tpu_sc as plsc

N = 10000
E = 160000
G = 128
EPS = 1e-5
NCLS = 10

QF = 128
NQ = 4
F = QF * NQ
R = 1000
NT = N // R

NCORES = 2
NSUB = 16
CH = 128
NCHT = E // CH
SCR = 10
SR = NCHT // SCR
TMAX = (SR + NSUB - 1) // NSUB
CPWF = (NCHT + NSUB - 1) // NSUB
NEXTRA = NCHT - (CPWF - 1) * NSUB
_DIAG_NO_SCATTER = False
_DIAG_NO_GATHER = False
NBUF = 3
IDXB = 5
RCH = 80
NRCH = N // RCH
RITER = (NRCH + NSUB - 1) // NSUB


def _deg_body(dst3_hbm, deg_hbm, idx_v, ones_v, row_v, acc_sh):
    c = lax.axis_index("c")
    s = lax.axis_index("s")

    @pl.when(c == 0)
    def _():
        one16 = jnp.full((16,), 1.0, jnp.float32)
        for i in range(CH // 16):
            ones_v[pl.ds(i * 16, 16)] = one16
        for i in range(RCH // 16):
            row_v[pl.ds(i * 16, 16)] = one16

        def init_body(k, _):
            ch = s + k * NSUB

            @pl.when(ch < NRCH)
            def _():
                pltpu.sync_copy(row_v, acc_sh.at[pl.ds(ch * RCH, RCH)])
            return _

        lax.fori_loop(0, RITER, init_body, None)
        plsc.subcore_barrier()

        for t in range(TMAX):
            sup = s + t * NSUB

            @pl.when(sup < SR)
            def _():
                pltpu.sync_copy(dst3_hbm.at[pl.ds(sup, 1), :, :], idx_v)
                for i in range(SCR):
                    pltpu.sync_copy(ones_v, acc_sh.at[idx_v.at[0, i]],
                                    add=True)
        plsc.subcore_barrier()

        def out_body(k, _):
            ch = s + k * NSUB

            @pl.when(ch < NRCH)
            def _():
                pltpu.sync_copy(acc_sh.at[pl.ds(ch * RCH, RCH)], row_v)
                pltpu.sync_copy(row_v, deg_hbm.at[pl.ds(ch * RCH, RCH)])
            return _

        lax.fori_loop(0, RITER, out_body, None)


def _agg_body(y_hbm, src_hbm, dst_hbm, agg_hbm,
              isrc_b, idst_b, stage_v,
              acc_sh, semg, sems, semi):
    c = lax.axis_index("c")
    s = lax.axis_index("s")
    nch = jnp.where(s < NEXTRA, CPWF, CPWF - 1)

    nrit = jnp.where(s < NRCH - (RITER - 1) * NSUB, RITER, RITER - 1)

    def _ring_rows(ext_ref, riter, to_acc):
        def slot(b):
            return stage_v.at[b, pl.ds(0, RCH), :]

        def rows(r, ref):
            ch = s + r * NSUB
            return ref.at[pl.ds(ch * RCH, RCH), :]

        def d_in(r, b):
            if to_acc:
                return pltpu.make_async_copy(rows(r, ext_ref), slot(b),
                                             semg.at[b])
            return pltpu.make_async_copy(rows(r, acc_sh), slot(b),
                                         semg.at[b])

        def d_out(r, b):
            if to_acc:
                return pltpu.make_async_copy(slot(b), rows(r, acc_sh),
                                             sems.at[b])
            return pltpu.make_async_copy(slot(b), rows(r, ext_ref),
                                         sems.at[b])

        def body(r, _):
            b = lax.rem(r, NBUF)

            @pl.when(r >= NBUF)
            def _():
                d_out(r - NBUF, b).wait()

            d_in(r, b).start()

            @pl.when(r >= 1)
            def _():
                bp = lax.rem(r - 1, NBUF)
                d_in(r - 1, bp).wait()
                d_out(r - 1, bp).start()
            return _

        lax.fori_loop(0, riter, body, None)
        rl = riter - 1
        bl = lax.rem(rl, NBUF)
        d_in(rl, bl).wait()
        d_out(rl, bl).start()
        for d in range(NBUF):
            rr = rl - d

            @pl.when(rr >= 0)
            def _():
                d_out(rr, lax.rem(rr, NBUF)).wait()

    for qi in range(NQ // NCORES):
        q = c * (NQ // NCORES) + qi
        yq = y_hbm.at[q]

        _ring_rows(yq, nrit, to_acc=True)
        plsc.subcore_barrier()

        def i_descs(k, b):
            off = (s + k * NSUB) * CH
            return (pltpu.make_async_copy(src_hbm.at[pl.ds(off, CH)],
                                          isrc_b.at[b], semi.at[b]),
                    pltpu.make_async_copy(dst_hbm.at[pl.ds(off, CH)],
                                          idst_b.at[b], semi.at[b]))

        def i_start(k, b):
            d0, d1 = i_descs(k, b)
            d0.start()
            d1.start()

        def i_wait(k, b):
            d0, d1 = i_descs(k, b)
            d0.wait()
            d1.wait()

        def g_desc(k, b):
            return pltpu.make_async_copy(
                yq.at[isrc_b.at[lax.rem(k, IDXB)]], stage_v.at[b],
                semg.at[b])

        def s_dst(k):
            return acc_sh.at[idst_b.at[lax.rem(k, IDXB)]]

        def s_wait(k, b):
            pltpu.make_async_copy(stage_v.at[b], s_dst(k), sems.at[b]).wait()

        def s_start(k, b):
            pltpu.async_copy(stage_v.at[b], s_dst(k), sems.at[b], add=True)

        i_start(0, 0)
        i_start(1, 1)

        def edge_body(k, _):
            b = lax.rem(k, NBUF)

            if not _DIAG_NO_SCATTER:
                @pl.when(k >= NBUF)
                def _():
                    s_wait(k - NBUF, b)

            @pl.when(k + 2 < nch)
            def _():
                i_start(k + 2, lax.rem(k + 2, IDXB))

            i_wait(k, lax.rem(k, IDXB))
            if not _DIAG_NO_GATHER:
                g_desc(k, b).start()

            @pl.when(k >= 1)
            def _():
                bp = lax.rem(k - 1, NBUF)
                if not _DIAG_NO_GATHER:
                    g_desc(k - 1, bp).wait()
                if not _DIAG_NO_SCATTER:
                    s_start(k - 1, bp)
            return _

        lax.fori_loop(0, nch, edge_body, None)
        kl = nch - 1
        bl = lax.rem(kl, NBUF)
        if not _DIAG_NO_GATHER:
            g_desc(kl, bl).wait()
        if not _DIAG_NO_SCATTER:
            s_start(kl, bl)
            for d in range(NBUF):
                kk = kl - d
                s_wait(kk, lax.rem(kk, NBUF))

        plsc.subcore_barrier()

        _ring_rows(agg_hbm.at[q], nrit, to_acc=False)


@functools.lru_cache(maxsize=None)
def _sc_kernels():
    mesh = plsc.VectorSubcoreMesh(
        core_axis_name="c", subcore_axis_name="s",
        num_cores=NCORES, num_subcores=NSUB)
    deg = pl.kernel(
        _deg_body,
        out_type=jax.ShapeDtypeStruct((N,), jnp.float32),
        mesh=mesh,
        scratch_types=[
            pltpu.VMEM((1, SCR, CH), jnp.int32),
            pltpu.VMEM((CH,), jnp.float32),
            pltpu.VMEM((RCH,), jnp.float32),
            pltpu.VMEM_SHARED((N,), jnp.float32),
        ],
    )
    agg = pl.kernel(
        _agg_body,
        out_type=jax.ShapeDtypeStruct((NQ, N, QF), jnp.float32),
        mesh=mesh,
        compiler_params=pltpu.CompilerParams(use_tc_tiling_on_sc=False),
        scratch_types=[
            pltpu.VMEM((IDXB, CH), jnp.int32),
            pltpu.VMEM((IDXB, CH), jnp.int32),
            pltpu.VMEM((NBUF, CH, QF), jnp.float32),
            pltpu.VMEM_SHARED((N, QF), jnp.float32),
            pltpu.SemaphoreType.DMA((NBUF,)),
            pltpu.SemaphoreType.DMA((NBUF,)),
            pltpu.SemaphoreType.DMA((IDXB,)),
        ],
    )
    return deg, agg


def _tc0_body(h_ref, w_ref, deg_ref, y_ref, dinv_ref):
    dinv = lax.rsqrt(deg_ref[...])
    dinv_ref[...] = dinv
    xw = jnp.dot(h_ref[...].astype(jnp.bfloat16),
                 w_ref[...].astype(jnp.bfloat16),
                 preferred_element_type=jnp.float32)
    y = xw * dinv
    for q in range(NQ):
        y_ref[q] = y[:, q * QF:(q + 1) * QF]


def _tc0(h, w0, deg):
    return pl.pallas_call(
        _tc0_body,
        grid=(NT,),
        in_specs=[
            pl.BlockSpec((R, 256), lambda j: (j, 0)),
            pl.BlockSpec((256, F), lambda j: (0, 0)),
            pl.BlockSpec((R, 1), lambda j: (j, 0)),
        ],
        out_specs=[
            pl.BlockSpec((NQ, R, QF), lambda j: (0, j, 0)),
            pl.BlockSpec((R, 1), lambda j: (j, 0)),
        ],
        out_shape=[
            jax.ShapeDtypeStruct((NQ, N, QF), jnp.float32),
            jax.ShapeDtypeStruct((N, 1), jnp.float32),
        ],
    )(h, w0, deg)


def _accum_stats(agg_ref, dinv, s_acc, ss_acc, j):
    @pl.when(j == 0)
    def _():
        s_acc[...] = jnp.zeros((NQ, QF), jnp.float32)
        ss_acc[...] = jnp.zeros((NQ, QF), jnp.float32)

    for q in range(NQ):
        t = agg_ref[q] * dinv
        s_acc[q, :] += jnp.sum(t, axis=0)
        ss_acc[q, :] += jnp.sum(t * t, axis=0)


def _bn_tile(agg_ref, dinv, gb, s_acc, ss_acc):
    mean = s_acc[...] * (1.0 / N)
    var = ss_acc[...] * (1.0 / N) - mean * mean
    rstd = lax.rsqrt(var + EPS)
    cols = []
    for q in range(NQ):
        t = agg_ref[q] * dinv
        cols.append((t - mean[q]) * rstd[q] * gb[q] + gb[NQ + q])
    return jnp.maximum(jnp.concatenate(cols, axis=1), 0.0)


def _make_tcmid(residual):
    def body(agg_ref, dinv_ref, gb_ref, *rest):
        if residual:
            xp_ref, w_ref, x_ref, y_ref, s_acc, ss_acc = rest
        else:
            w_ref, x_ref, y_ref, s_acc, ss_acc = rest
        p = pl.program_id(0)
        j = pl.program_id(1)
        dinv = dinv_ref[...]

        @pl.when(p == 0)
        def _():
            _accum_stats(agg_ref, dinv, s_acc, ss_acc, j)

        @pl.when(p == 1)
        def _():
            x = _bn_tile(agg_ref, dinv, gb_ref[...], s_acc, ss_acc)
            if residual:
                x = x + xp_ref[...]
            x_ref[...] = x
            y = jnp.dot(x.astype(jnp.bfloat16),
                        w_ref[...].astype(jnp.bfloat16),
                        preferred_element_type=jnp.float32) * dinv
            for q in range(NQ):
                y_ref[q] = y[:, q * QF:(q + 1) * QF]

    in_specs = [
        pl.BlockSpec((NQ, R, QF), lambda p, j: (0, j, 0)),
        pl.BlockSpec((R, 1), lambda p, j: (j, 0)),
        pl.BlockSpec((2 * NQ, QF), lambda p, j: (0, 0)),
    ]
    if residual:
        in_specs.append(
            pl.BlockSpec((R, F), lambda p, j: (jnp.where(p == 1, j, 0), 0)))
    in_specs.append(pl.BlockSpec((F, F), lambda p, j: (0, 0)))

    def run(*args):
        return pl.pallas_call(
            body,
            grid=(2, NT),
            in_specs=in_specs,
            out_specs=[
                pl.BlockSpec((R, F), lambda p, j: (jnp.where(p == 1, j, 0), 0)),
                pl.BlockSpec((NQ, R, QF),
                             lambda p, j: (0, jnp.where(p == 1, j, 0), 0)),
            ],
            out_shape=[
                jax.ShapeDtypeStruct((N, F), jnp.float32),
                jax.ShapeDtypeStruct((NQ, N, QF), jnp.float32),
            ],
            scratch_shapes=[
                pltpu.VMEM((NQ, QF), jnp.float32),
                pltpu.VMEM((NQ, QF), jnp.float32),
            ],
        )(*args)

    return run


_tcmid_nores = _make_tcmid(False)
_tcmid_res = _make_tcmid(True)


def _tcfinal_body(agg_ref, dinv_ref, gb_ref, xp_ref, batch_ref,
                  wm_ref, bm_ref, out_ref, s_acc, ss_acc, sums_acc, cnt_acc):
    p = pl.program_id(0)
    j = pl.program_id(1)
    dinv = dinv_ref[...]

    @pl.when(p == 0)
    def _():
        _accum_stats(agg_ref, dinv, s_acc, ss_acc, j)

    @pl.when(p == 1)
    def _():
        @pl.when(j == 0)
        def _():
            sums_acc[...] = jnp.zeros((G, F), jnp.float32)
            cnt_acc[...] = jnp.zeros((G, 1), jnp.float32)

        x = _bn_tile(agg_ref, dinv, gb_ref[...], s_acc, ss_acc)
        x = x + xp_ref[...]
        b = batch_ref[...].reshape(1, R)
        gids = lax.broadcasted_iota(jnp.int32, (G, R), 0)
        ind = (gids == b).astype(jnp.float32)
        sums_acc[...] += jnp.dot(ind, x, preferred_element_type=jnp.float32)
        cnt_acc[...] += jnp.sum(ind, axis=1, keepdims=True)

        @pl.when(j == NT - 1)
        def _():
            hg = sums_acc[...] / jnp.maximum(cnt_acc[...], 1.0)
            out_ref[...] = (
                jnp.dot(hg, wm_ref[...], preferred_element_type=jnp.float32)
                + bm_ref[0:1, :])


def _tcfinal(agg, dinv, gb, xp, batch3, wm, bm8):
    return pl.pallas_call(
        _tcfinal_body,
        grid=(2, NT),
        in_specs=[
            pl.BlockSpec((NQ, R, QF), lambda p, j: (0, j, 0)),
            pl.BlockSpec((R, 1), lambda p, j: (j, 0)),
            pl.BlockSpec((2 * NQ, QF), lambda p, j: (0, 0)),
            pl.BlockSpec((R, F), lambda p, j: (jnp.where(p == 1, j, 0), 0)),
            pl.BlockSpec((1, 1, R),
                         lambda p, j: (jnp.where(p == 1, j, 0), 0, 0)),
            pl.BlockSpec((F, NCLS), lambda p, j: (0, 0)),
            pl.BlockSpec((8, NCLS), lambda p, j: (0, 0)),
        ],
        out_specs=pl.BlockSpec((G, NCLS), lambda p, j: (0, 0)),
        out_shape=jax.ShapeDtypeStruct((G, NCLS), jnp.float32),
        scratch_shapes=[
            pltpu.VMEM((NQ, QF), jnp.float32),
            pltpu.VMEM((NQ, QF), jnp.float32),
            pltpu.VMEM((G, F), jnp.float32),
            pltpu.VMEM((G, 1), jnp.float32),
        ],
    )(agg, dinv, gb, xp, batch3, wm, bm8)


def kernel(h, edge_index, batch, e, W0, b0, g0, be0, W1, b1, g1, be1,
           W2, b2, g2, be2, W3, b3, g3, be3, Wm, bm):
    src = edge_index[0]
    dst = edge_index[1]
    dst3 = dst.reshape(SR, SCR, CH)
    _deg_kernel, _agg_kernel = _sc_kernels()

    deg = _deg_kernel(dst3).reshape(N, 1)
    y, dinv = _tc0(h, W0, deg)

    gbs = [jnp.concatenate([g.reshape(NQ, QF), be.reshape(NQ, QF)], axis=0)
           for g, be in ((g0, be0), (g1, be1), (g2, be2), (g3, be3))]
    ws = [W1, W2, W3]

    x = None
    for i in range(3):
        agg = _agg_kernel(y, src, dst)
        if i == 0:
            x, y = _tcmid_nores(agg, dinv, gbs[i], ws[i])
        else:
            x, y = _tcmid_res(agg, dinv, gbs[i], x, ws[i])

    agg = _agg_kernel(y, src, dst)
    batch3 = batch.reshape(NT, 1, R)
    bm8 = jnp.broadcast_to(bm.reshape(1, NCLS), (8, NCLS))
    return _tcfinal(agg, dinv, gbs[3], x, batch3, Wm, bm8)


# speedup vs baseline: 1.0011x; 1.0011x over previous
import functools

import jax
import jax.numpy as jnp
from jax import lax
from jax.experimental import pallas as pl
from jax.experimental.pallas import tpu as pltpu
from jax.experimental.pallas import tpu_sc as plsc

N = 10000
E = 160000
G = 128
EPS = 1e-5
NCLS = 10

QF = 128
NQ = 4
F = QF * NQ
R = 1000
NT = N // R

NCORES = 2
NSUB = 16
CH = 128
NCHT = E // CH
SCR = 10
SR = NCHT // SCR
TMAX = (SR + NSUB - 1) // NSUB
CPWF = (NCHT + NSUB - 1) // NSUB
NEXTRA = NCHT - (CPWF - 1) * NSUB
_DIAG_NO_SCATTER = False
_DIAG_NO_GATHER = False
NBUF = 3
IDXB = 5
RCH = 80
NRCH = N // RCH
RITER = (NRCH + NSUB - 1) // NSUB


def _deg_body(dst_hbm, deg_hbm, idx_b, ones_v, row_v, acc_sh, semi, sems):
    c = lax.axis_index("c")
    s = lax.axis_index("s")

    @pl.when(c == 0)
    def _():
        one16 = jnp.full((16,), 1.0, jnp.float32)
        for i in range(CH // 16):
            ones_v[pl.ds(i * 16, 16)] = one16
        for i in range(RCH // 16):
            row_v[pl.ds(i * 16, 16)] = one16

        def init_body(k, _):
            ch = s + k * NSUB

            @pl.when(ch < NRCH)
            def _():
                pltpu.sync_copy(row_v, acc_sh.at[pl.ds(ch * RCH, RCH)])
            return _

        lax.fori_loop(0, RITER, init_body, None)
        plsc.subcore_barrier()

        nch = jnp.where(s < NEXTRA, CPWF, CPWF - 1)

        def i_desc(k, b):
            off = (s + k * NSUB) * CH
            return pltpu.make_async_copy(dst_hbm.at[pl.ds(off, CH)],
                                         idx_b.at[b], semi.at[b])

        def s_desc(b):
            return pltpu.make_async_copy(ones_v, acc_sh.at[idx_b.at[b]],
                                         sems.at[b])

        def s_start(b):
            pltpu.async_copy(ones_v, acc_sh.at[idx_b.at[b]], sems.at[b],
                             add=True)

        i_desc(0, 0).start()
        i_desc(1, 1).start()

        def body(k, _):
            b = lax.rem(k, IDXB)

            @pl.when(k >= 3)
            def _():
                s_desc(lax.rem(k - 3, IDXB)).wait()

            @pl.when(k + 2 < nch)
            def _():
                i_desc(k + 2, lax.rem(k + 2, IDXB)).start()

            i_desc(k, b).wait()
            s_start(b)
            return _

        lax.fori_loop(0, nch, body, None)
        for d in range(3):
            @pl.when(nch - 1 - d >= 0)
            def _():
                s_desc(lax.rem(nch - 1 - d, IDXB)).wait()
        plsc.subcore_barrier()

        def out_body(k, _):
            ch = s + k * NSUB

            @pl.when(ch < NRCH)
            def _():
                pltpu.sync_copy(acc_sh.at[pl.ds(ch * RCH, RCH)], row_v)
                pltpu.sync_copy(row_v, deg_hbm.at[pl.ds(ch * RCH, RCH)])
            return _

        lax.fori_loop(0, RITER, out_body, None)


def _agg_body(y_hbm, src_hbm, dst_hbm, agg_hbm,
              isrc_b, idst_b, stage_v,
              acc_sh, semg, sems, semi):
    c = lax.axis_index("c")
    s = lax.axis_index("s")
    nch = jnp.where(s < NEXTRA, CPWF, CPWF - 1)

    nrit = jnp.where(s < NRCH - (RITER - 1) * NSUB, RITER, RITER - 1)

    def _ring_rows(ext_ref, riter, to_acc):
        def slot(b):
            return stage_v.at[b, pl.ds(0, RCH), :]

        def rows(r, ref):
            ch = s + r * NSUB
            return ref.at[pl.ds(ch * RCH, RCH), :]

        def d_in(r, b):
            if to_acc:
                return pltpu.make_async_copy(rows(r, ext_ref), slot(b),
                                             semg.at[b])
            return pltpu.make_async_copy(rows(r, acc_sh), slot(b),
                                         semg.at[b])

        def d_out(r, b):
            if to_acc:
                return pltpu.make_async_copy(slot(b), rows(r, acc_sh),
                                             sems.at[b])
            return pltpu.make_async_copy(slot(b), rows(r, ext_ref),
                                         sems.at[b])

        def body(r, _):
            b = lax.rem(r, NBUF)

            @pl.when(r >= NBUF)
            def _():
                d_out(r - NBUF, b).wait()

            d_in(r, b).start()

            @pl.when(r >= 1)
            def _():
                bp = lax.rem(r - 1, NBUF)
                d_in(r - 1, bp).wait()
                d_out(r - 1, bp).start()
            return _

        lax.fori_loop(0, riter, body, None)
        rl = riter - 1
        bl = lax.rem(rl, NBUF)
        d_in(rl, bl).wait()
        d_out(rl, bl).start()
        for d in range(NBUF):
            rr = rl - d

            @pl.when(rr >= 0)
            def _():
                d_out(rr, lax.rem(rr, NBUF)).wait()

    for qi in range(NQ // NCORES):
        q = c * (NQ // NCORES) + qi
        yq = y_hbm.at[q]

        _ring_rows(yq, nrit, to_acc=True)
        plsc.subcore_barrier()

        def i_descs(k, b):
            off = (s + k * NSUB) * CH
            return (pltpu.make_async_copy(src_hbm.at[pl.ds(off, CH)],
                                          isrc_b.at[b], semi.at[b]),
                    pltpu.make_async_copy(dst_hbm.at[pl.ds(off, CH)],
                                          idst_b.at[b], semi.at[b]))

        def i_start(k, b):
            d0, d1 = i_descs(k, b)
            d0.start()
            d1.start()

        def i_wait(k, b):
            d0, d1 = i_descs(k, b)
            d0.wait()
            d1.wait()

        def g_desc(k, b):
            return pltpu.make_async_copy(
                yq.at[isrc_b.at[lax.rem(k, IDXB)]], stage_v.at[b],
                semg.at[b])

        def s_dst(k):
            return acc_sh.at[idst_b.at[lax.rem(k, IDXB)]]

        def s_wait(k, b):
            pltpu.make_async_copy(stage_v.at[b], s_dst(k), sems.at[b]).wait()

        def s_start(k, b):
            pltpu.async_copy(stage_v.at[b], s_dst(k), sems.at[b], add=True)

        i_start(0, 0)
        i_start(1, 1)

        def edge_body(k, _):
            b = lax.rem(k, NBUF)

            if not _DIAG_NO_SCATTER:
                @pl.when(k >= NBUF)
                def _():
                    s_wait(k - NBUF, b)

            @pl.when(k + 2 < nch)
            def _():
                i_start(k + 2, lax.rem(k + 2, IDXB))

            i_wait(k, lax.rem(k, IDXB))
            if not _DIAG_NO_GATHER:
                g_desc(k, b).start()

            @pl.when(k >= 1)
            def _():
                bp = lax.rem(k - 1, NBUF)
                if not _DIAG_NO_GATHER:
                    g_desc(k - 1, bp).wait()
                if not _DIAG_NO_SCATTER:
                    s_start(k - 1, bp)
            return _

        lax.fori_loop(0, nch, edge_body, None)
        kl = nch - 1
        bl = lax.rem(kl, NBUF)
        if not _DIAG_NO_GATHER:
            g_desc(kl, bl).wait()
        if not _DIAG_NO_SCATTER:
            s_start(kl, bl)
            for d in range(NBUF):
                kk = kl - d
                s_wait(kk, lax.rem(kk, NBUF))

        plsc.subcore_barrier()

        _ring_rows(agg_hbm.at[q], nrit, to_acc=False)


@functools.lru_cache(maxsize=None)
def _sc_kernels():
    mesh = plsc.VectorSubcoreMesh(
        core_axis_name="c", subcore_axis_name="s",
        num_cores=NCORES, num_subcores=NSUB)
    deg = pl.kernel(
        _deg_body,
        out_type=jax.ShapeDtypeStruct((N,), jnp.float32),
        mesh=mesh,
        scratch_types=[
            pltpu.VMEM((IDXB, CH), jnp.int32),
            pltpu.VMEM((CH,), jnp.float32),
            pltpu.VMEM((RCH,), jnp.float32),
            pltpu.VMEM_SHARED((N,), jnp.float32),
            pltpu.SemaphoreType.DMA((IDXB,)),
            pltpu.SemaphoreType.DMA((IDXB,)),
        ],
    )
    agg = pl.kernel(
        _agg_body,
        out_type=jax.ShapeDtypeStruct((NQ, N, QF), jnp.float32),
        mesh=mesh,
        compiler_params=pltpu.CompilerParams(use_tc_tiling_on_sc=False),
        scratch_types=[
            pltpu.VMEM((IDXB, CH), jnp.int32),
            pltpu.VMEM((IDXB, CH), jnp.int32),
            pltpu.VMEM((NBUF, CH, QF), jnp.float32),
            pltpu.VMEM_SHARED((N, QF), jnp.float32),
            pltpu.SemaphoreType.DMA((NBUF,)),
            pltpu.SemaphoreType.DMA((NBUF,)),
            pltpu.SemaphoreType.DMA((IDXB,)),
        ],
    )
    return deg, agg


def _tc0_body(h_ref, w_ref, deg_ref, y_ref, dinv_ref):
    dinv = lax.rsqrt(deg_ref[...])
    dinv_ref[...] = dinv
    xw = jnp.dot(h_ref[...], w_ref[...], preferred_element_type=jnp.float32)
    y = xw * dinv
    for q in range(NQ):
        y_ref[q] = y[:, q * QF:(q + 1) * QF]


def _tc0(h, w0, deg):
    return pl.pallas_call(
        _tc0_body,
        grid=(NT,),
        in_specs=[
            pl.BlockSpec((R, 256), lambda j: (j, 0)),
            pl.BlockSpec((256, F), lambda j: (0, 0)),
            pl.BlockSpec((R, 1), lambda j: (j, 0)),
        ],
        out_specs=[
            pl.BlockSpec((NQ, R, QF), lambda j: (0, j, 0)),
            pl.BlockSpec((R, 1), lambda j: (j, 0)),
        ],
        out_shape=[
            jax.ShapeDtypeStruct((NQ, N, QF), jnp.float32),
            jax.ShapeDtypeStruct((N, 1), jnp.float32),
        ],
    )(h, w0, deg)


def _accum_stats(agg_ref, dinv, s_acc, ss_acc, j):
    @pl.when(j == 0)
    def _():
        s_acc[...] = jnp.zeros((NQ, QF), jnp.float32)
        ss_acc[...] = jnp.zeros((NQ, QF), jnp.float32)

    for q in range(NQ):
        t = agg_ref[q] * dinv
        s_acc[q, :] += jnp.sum(t, axis=0)
        ss_acc[q, :] += jnp.sum(t * t, axis=0)


def _bn_tile(agg_ref, dinv, gb, s_acc, ss_acc):
    mean = s_acc[...] * (1.0 / N)
    var = ss_acc[...] * (1.0 / N) - mean * mean
    rstd = lax.rsqrt(var + EPS)
    cols = []
    for q in range(NQ):
        t = agg_ref[q] * dinv
        cols.append((t - mean[q]) * rstd[q] * gb[q] + gb[NQ + q])
    return jnp.maximum(jnp.concatenate(cols, axis=1), 0.0)


def _make_tcmid(residual):
    def body(agg_ref, dinv_ref, gb_ref, *rest):
        if residual:
            xp_ref, w_ref, x_ref, y_ref, s_acc, ss_acc = rest
        else:
            w_ref, x_ref, y_ref, s_acc, ss_acc = rest
        p = pl.program_id(0)
        j = pl.program_id(1)
        dinv = dinv_ref[...]

        @pl.when(p == 0)
        def _():
            _accum_stats(agg_ref, dinv, s_acc, ss_acc, j)

        @pl.when(p == 1)
        def _():
            x = _bn_tile(agg_ref, dinv, gb_ref[...], s_acc, ss_acc)
            if residual:
                x = x + xp_ref[...]
            x_ref[...] = x
            y = jnp.dot(x, w_ref[...],
                        preferred_element_type=jnp.float32) * dinv
            for q in range(NQ):
                y_ref[q] = y[:, q * QF:(q + 1) * QF]

    in_specs = [
        pl.BlockSpec((NQ, R, QF), lambda p, j: (0, j, 0)),
        pl.BlockSpec((R, 1), lambda p, j: (j, 0)),
        pl.BlockSpec((2 * NQ, QF), lambda p, j: (0, 0)),
    ]
    if residual:
        in_specs.append(
            pl.BlockSpec((R, F), lambda p, j: (jnp.where(p == 1, j, 0), 0)))
    in_specs.append(pl.BlockSpec((F, F), lambda p, j: (0, 0)))

    def run(*args):
        return pl.pallas_call(
            body,
            grid=(2, NT),
            in_specs=in_specs,
            out_specs=[
                pl.BlockSpec((R, F), lambda p, j: (jnp.where(p == 1, j, 0), 0)),
                pl.BlockSpec((NQ, R, QF),
                             lambda p, j: (0, jnp.where(p == 1, j, 0), 0)),
            ],
            out_shape=[
                jax.ShapeDtypeStruct((N, F), jnp.float32),
                jax.ShapeDtypeStruct((NQ, N, QF), jnp.float32),
            ],
            scratch_shapes=[
                pltpu.VMEM((NQ, QF), jnp.float32),
                pltpu.VMEM((NQ, QF), jnp.float32),
            ],
        )(*args)

    return run


_tcmid_nores = _make_tcmid(False)
_tcmid_res = _make_tcmid(True)


def _tcfinal_body(agg_ref, dinv_ref, gb_ref, xp_ref, batch_ref,
                  wm_ref, bm_ref, out_ref, s_acc, ss_acc, sums_acc, cnt_acc):
    p = pl.program_id(0)
    j = pl.program_id(1)
    dinv = dinv_ref[...]

    @pl.when(p == 0)
    def _():
        _accum_stats(agg_ref, dinv, s_acc, ss_acc, j)

    @pl.when(p == 1)
    def _():
        @pl.when(j == 0)
        def _():
            sums_acc[...] = jnp.zeros((G, F), jnp.float32)
            cnt_acc[...] = jnp.zeros((G, 1), jnp.float32)

        x = _bn_tile(agg_ref, dinv, gb_ref[...], s_acc, ss_acc)
        x = x + xp_ref[...]
        b = batch_ref[...].reshape(1, R)
        gids = lax.broadcasted_iota(jnp.int32, (G, R), 0)
        ind = (gids == b).astype(jnp.float32)
        sums_acc[...] += jnp.dot(ind, x, preferred_element_type=jnp.float32)
        cnt_acc[...] += jnp.sum(ind, axis=1, keepdims=True)

        @pl.when(j == NT - 1)
        def _():
            hg = sums_acc[...] / jnp.maximum(cnt_acc[...], 1.0)
            out_ref[...] = (
                jnp.dot(hg, wm_ref[...], preferred_element_type=jnp.float32)
                + bm_ref[0:1, :])


def _tcfinal(agg, dinv, gb, xp, batch3, wm, bm8):
    return pl.pallas_call(
        _tcfinal_body,
        grid=(2, NT),
        in_specs=[
            pl.BlockSpec((NQ, R, QF), lambda p, j: (0, j, 0)),
            pl.BlockSpec((R, 1), lambda p, j: (j, 0)),
            pl.BlockSpec((2 * NQ, QF), lambda p, j: (0, 0)),
            pl.BlockSpec((R, F), lambda p, j: (jnp.where(p == 1, j, 0), 0)),
            pl.BlockSpec((1, 1, R),
                         lambda p, j: (jnp.where(p == 1, j, 0), 0, 0)),
            pl.BlockSpec((F, NCLS), lambda p, j: (0, 0)),
            pl.BlockSpec((8, NCLS), lambda p, j: (0, 0)),
        ],
        out_specs=pl.BlockSpec((G, NCLS), lambda p, j: (0, 0)),
        out_shape=jax.ShapeDtypeStruct((G, NCLS), jnp.float32),
        scratch_shapes=[
            pltpu.VMEM((NQ, QF), jnp.float32),
            pltpu.VMEM((NQ, QF), jnp.float32),
            pltpu.VMEM((G, F), jnp.float32),
            pltpu.VMEM((G, 1), jnp.float32),
        ],
    )(agg, dinv, gb, xp, batch3, wm, bm8)


def kernel(h, edge_index, batch, e, W0, b0, g0, be0, W1, b1, g1, be1,
           W2, b2, g2, be2, W3, b3, g3, be3, Wm, bm):
    src = edge_index[0]
    dst = edge_index[1]
    _deg_kernel, _agg_kernel = _sc_kernels()

    deg = _deg_kernel(dst).reshape(N, 1)
    y, dinv = _tc0(h, W0, deg)

    gbs = [jnp.concatenate([g.reshape(NQ, QF), be.reshape(NQ, QF)], axis=0)
           for g, be in ((g0, be0), (g1, be1), (g2, be2), (g3, be3))]
    ws = [W1, W2, W3]

    x = None
    for i in range(3):
        agg = _agg_kernel(y, src, dst)
        if i == 0:
            x, y = _tcmid_nores(agg, dinv, gbs[i], ws[i])
        else:
            x, y = _tcmid_res(agg, dinv, gbs[i], x, ws[i])

    agg = _agg_kernel(y, src, dst)
    batch3 = batch.reshape(NT, 1, R)
    bm8 = jnp.broadcast_to(bm.reshape(1, NCLS), (8, NCLS))
    return _tcfinal(agg, dinv, gbs[3], x, batch3, Wm, bm8)


# speedup vs baseline: 1.0699x; 1.0687x over previous
import functools

import jax
import jax.numpy as jnp
from jax import lax
from jax.experimental import pallas as pl
from jax.experimental.pallas import tpu as pltpu
from jax.experimental.pallas import tpu_sc as plsc

N = 10000
E = 160000
G = 128
EPS = 1e-5
NCLS = 10

QF = 128
NQ = 4
F = QF * NQ
R = 1000
NT = N // R

NCORES = 2
NSUB = 16
CH = 128
NCHT = E // CH
SCR = 10
SR = NCHT // SCR
TMAX = (SR + NSUB - 1) // NSUB
CPWF = (NCHT + NSUB - 1) // NSUB
NEXTRA = NCHT - (CPWF - 1) * NSUB
_DIAG_NO_SCATTER = False
_DIAG_NO_GATHER = False
NBUF = 3
GLAG = 2
IDXB = 5
RCH = 80
NRCH = N // RCH
RITER = (NRCH + NSUB - 1) // NSUB


def _deg_body(dst_hbm, deg_hbm, idx_b, ones_v, row_v, acc_sh, semi, sems):
    c = lax.axis_index("c")
    s = lax.axis_index("s")

    @pl.when(c == 0)
    def _():
        one16 = jnp.full((16,), 1.0, jnp.float32)
        for i in range(CH // 16):
            ones_v[pl.ds(i * 16, 16)] = one16
        for i in range(RCH // 16):
            row_v[pl.ds(i * 16, 16)] = one16

        def init_body(k, _):
            ch = s + k * NSUB

            @pl.when(ch < NRCH)
            def _():
                pltpu.sync_copy(row_v, acc_sh.at[pl.ds(ch * RCH, RCH)])
            return _

        lax.fori_loop(0, RITER, init_body, None)
        plsc.subcore_barrier()

        nch = jnp.where(s < NEXTRA, CPWF, CPWF - 1)

        def i_desc(k, b):
            off = (s + k * NSUB) * CH
            return pltpu.make_async_copy(dst_hbm.at[pl.ds(off, CH)],
                                         idx_b.at[b], semi.at[b])

        def s_desc(b):
            return pltpu.make_async_copy(ones_v, acc_sh.at[idx_b.at[b]],
                                         sems.at[b])

        def s_start(b):
            pltpu.async_copy(ones_v, acc_sh.at[idx_b.at[b]], sems.at[b],
                             add=True)

        i_desc(0, 0).start()
        i_desc(1, 1).start()

        def body(k, _):
            b = lax.rem(k, IDXB)

            @pl.when(k >= 3)
            def _():
                s_desc(lax.rem(k - 3, IDXB)).wait()

            @pl.when(k + 2 < nch)
            def _():
                i_desc(k + 2, lax.rem(k + 2, IDXB)).start()

            i_desc(k, b).wait()
            s_start(b)
            return _

        lax.fori_loop(0, nch, body, None)
        for d in range(3):
            @pl.when(nch - 1 - d >= 0)
            def _():
                s_desc(lax.rem(nch - 1 - d, IDXB)).wait()
        plsc.subcore_barrier()

        def out_body(k, _):
            ch = s + k * NSUB

            @pl.when(ch < NRCH)
            def _():
                pltpu.sync_copy(acc_sh.at[pl.ds(ch * RCH, RCH)], row_v)
                pltpu.sync_copy(row_v, deg_hbm.at[pl.ds(ch * RCH, RCH)])
            return _

        lax.fori_loop(0, RITER, out_body, None)


def _agg_body(y_hbm, src_hbm, dst_hbm, agg_hbm,
              isrc_b, idst_b, stage_v,
              acc_sh, semg, sems, semi):
    c = lax.axis_index("c")
    s = lax.axis_index("s")
    nch = jnp.where(s < NEXTRA, CPWF, CPWF - 1)

    nrit = jnp.where(s < NRCH - (RITER - 1) * NSUB, RITER, RITER - 1)

    def _ring_rows(ext_ref, riter, to_acc):
        def slot(b):
            return stage_v.at[b, pl.ds(0, RCH), :]

        def rows(r, ref):
            ch = s + r * NSUB
            return ref.at[pl.ds(ch * RCH, RCH), :]

        def d_in(r, b):
            if to_acc:
                return pltpu.make_async_copy(rows(r, ext_ref), slot(b),
                                             semg.at[b])
            return pltpu.make_async_copy(rows(r, acc_sh), slot(b),
                                         semg.at[b])

        def d_out(r, b):
            if to_acc:
                return pltpu.make_async_copy(slot(b), rows(r, acc_sh),
                                             sems.at[b])
            return pltpu.make_async_copy(slot(b), rows(r, ext_ref),
                                         sems.at[b])

        def body(r, _):
            b = lax.rem(r, NBUF)

            @pl.when(r >= NBUF)
            def _():
                d_out(r - NBUF, b).wait()

            d_in(r, b).start()

            @pl.when(r >= 1)
            def _():
                bp = lax.rem(r - 1, NBUF)
                d_in(r - 1, bp).wait()
                d_out(r - 1, bp).start()
            return _

        lax.fori_loop(0, riter, body, None)
        rl = riter - 1
        bl = lax.rem(rl, NBUF)
        d_in(rl, bl).wait()
        d_out(rl, bl).start()
        for d in range(NBUF):
            rr = rl - d

            @pl.when(rr >= 0)
            def _():
                d_out(rr, lax.rem(rr, NBUF)).wait()

    for qi in range(NQ // NCORES):
        q = c * (NQ // NCORES) + qi
        yq = y_hbm.at[q]

        _ring_rows(yq, nrit, to_acc=True)
        plsc.subcore_barrier()

        def i_descs(k, b):
            off = (s + k * NSUB) * CH
            return (pltpu.make_async_copy(src_hbm.at[pl.ds(off, CH)],
                                          isrc_b.at[b], semi.at[b]),
                    pltpu.make_async_copy(dst_hbm.at[pl.ds(off, CH)],
                                          idst_b.at[b], semi.at[b]))

        def i_start(k, b):
            d0, d1 = i_descs(k, b)
            d0.start()
            d1.start()

        def i_wait(k, b):
            d0, d1 = i_descs(k, b)
            d0.wait()
            d1.wait()

        def g_desc(k, b):
            return pltpu.make_async_copy(
                yq.at[isrc_b.at[lax.rem(k, IDXB)]], stage_v.at[b],
                semg.at[b])

        def s_dst(k):
            return acc_sh.at[idst_b.at[lax.rem(k, IDXB)]]

        def s_wait(k, b):
            pltpu.make_async_copy(stage_v.at[b], s_dst(k), sems.at[b]).wait()

        def s_start(k, b):
            pltpu.async_copy(stage_v.at[b], s_dst(k), sems.at[b], add=True)

        i_start(0, 0)
        i_start(1, 1)

        def edge_body(k, _):
            b = lax.rem(k, NBUF)

            if not _DIAG_NO_SCATTER:
                @pl.when(k >= NBUF)
                def _():
                    s_wait(k - NBUF, b)

            @pl.when(k + 2 < nch)
            def _():
                i_start(k + 2, lax.rem(k + 2, IDXB))

            i_wait(k, lax.rem(k, IDXB))
            if not _DIAG_NO_GATHER:
                g_desc(k, b).start()

            @pl.when(k >= GLAG)
            def _():
                bp = lax.rem(k - GLAG, NBUF)
                if not _DIAG_NO_GATHER:
                    g_desc(k - GLAG, bp).wait()
                if not _DIAG_NO_SCATTER:
                    s_start(k - GLAG, bp)
            return _

        lax.fori_loop(0, nch, edge_body, None)
        for d in range(GLAG - 1, -1, -1):
            kk = nch - 1 - d
            bk = lax.rem(kk, NBUF)
            if not _DIAG_NO_GATHER:
                g_desc(kk, bk).wait()
            if not _DIAG_NO_SCATTER:
                s_start(kk, bk)
        if not _DIAG_NO_SCATTER:
            for d in range(NBUF):
                kk = nch - 1 - d
                s_wait(kk, lax.rem(kk, NBUF))

        plsc.subcore_barrier()

        _ring_rows(agg_hbm.at[q], nrit, to_acc=False)


@functools.lru_cache(maxsize=None)
def _sc_kernels():
    mesh = plsc.VectorSubcoreMesh(
        core_axis_name="c", subcore_axis_name="s",
        num_cores=NCORES, num_subcores=NSUB)
    deg = pl.kernel(
        _deg_body,
        out_type=jax.ShapeDtypeStruct((N,), jnp.float32),
        mesh=mesh,
        scratch_types=[
            pltpu.VMEM((IDXB, CH), jnp.int32),
            pltpu.VMEM((CH,), jnp.float32),
            pltpu.VMEM((RCH,), jnp.float32),
            pltpu.VMEM_SHARED((N,), jnp.float32),
            pltpu.SemaphoreType.DMA((IDXB,)),
            pltpu.SemaphoreType.DMA((IDXB,)),
        ],
    )
    agg = pl.kernel(
        _agg_body,
        out_type=jax.ShapeDtypeStruct((NQ, N, QF), jnp.float32),
        mesh=mesh,
        compiler_params=pltpu.CompilerParams(use_tc_tiling_on_sc=False),
        scratch_types=[
            pltpu.VMEM((IDXB, CH), jnp.int32),
            pltpu.VMEM((IDXB, CH), jnp.int32),
            pltpu.VMEM((NBUF, CH, QF), jnp.float32),
            pltpu.VMEM_SHARED((N, QF), jnp.float32),
            pltpu.SemaphoreType.DMA((NBUF,)),
            pltpu.SemaphoreType.DMA((NBUF,)),
            pltpu.SemaphoreType.DMA((IDXB,)),
        ],
    )
    return deg, agg


def _tc0_body(h_ref, w_ref, deg_ref, y_ref, dinv_ref):
    dinv = lax.rsqrt(deg_ref[...])
    dinv_ref[...] = dinv
    xw = jnp.dot(h_ref[...], w_ref[...], preferred_element_type=jnp.float32)
    y = xw * dinv
    for q in range(NQ):
        y_ref[q] = y[:, q * QF:(q + 1) * QF]


def _tc0(h, w0, deg):
    return pl.pallas_call(
        _tc0_body,
        grid=(NT,),
        in_specs=[
            pl.BlockSpec((R, 256), lambda j: (j, 0)),
            pl.BlockSpec((256, F), lambda j: (0, 0)),
            pl.BlockSpec((R, 1), lambda j: (j, 0)),
        ],
        out_specs=[
            pl.BlockSpec((NQ, R, QF), lambda j: (0, j, 0)),
            pl.BlockSpec((R, 1), lambda j: (j, 0)),
        ],
        out_shape=[
            jax.ShapeDtypeStruct((NQ, N, QF), jnp.float32),
            jax.ShapeDtypeStruct((N, 1), jnp.float32),
        ],
    )(h, w0, deg)


def _accum_stats(agg_ref, dinv, s_acc, ss_acc, j):
    @pl.when(j == 0)
    def _():
        s_acc[...] = jnp.zeros((NQ, QF), jnp.float32)
        ss_acc[...] = jnp.zeros((NQ, QF), jnp.float32)

    for q in range(NQ):
        t = agg_ref[q] * dinv
        s_acc[q, :] += jnp.sum(t, axis=0)
        ss_acc[q, :] += jnp.sum(t * t, axis=0)


def _bn_tile(agg_ref, dinv, gb, s_acc, ss_acc):
    mean = s_acc[...] * (1.0 / N)
    var = ss_acc[...] * (1.0 / N) - mean * mean
    rstd = lax.rsqrt(var + EPS)
    cols = []
    for q in range(NQ):
        t = agg_ref[q] * dinv
        cols.append((t - mean[q]) * rstd[q] * gb[q] + gb[NQ + q])
    return jnp.maximum(jnp.concatenate(cols, axis=1), 0.0)


def _make_tcmid(residual):
    def body(agg_ref, dinv_ref, gb_ref, *rest):
        if residual:
            xp_ref, w_ref, x_ref, y_ref, s_acc, ss_acc = rest
        else:
            w_ref, x_ref, y_ref, s_acc, ss_acc = rest
        p = pl.program_id(0)
        j = pl.program_id(1)
        dinv = dinv_ref[...]

        @pl.when(p == 0)
        def _():
            _accum_stats(agg_ref, dinv, s_acc, ss_acc, j)

        @pl.when(p == 1)
        def _():
            x = _bn_tile(agg_ref, dinv, gb_ref[...], s_acc, ss_acc)
            if residual:
                x = x + xp_ref[...]
            x_ref[...] = x
            y = jnp.dot(x, w_ref[...],
                        preferred_element_type=jnp.float32) * dinv
            for q in range(NQ):
                y_ref[q] = y[:, q * QF:(q + 1) * QF]

    in_specs = [
        pl.BlockSpec((NQ, R, QF), lambda p, j: (0, j, 0)),
        pl.BlockSpec((R, 1), lambda p, j: (j, 0)),
        pl.BlockSpec((2 * NQ, QF), lambda p, j: (0, 0)),
    ]
    if residual:
        in_specs.append(
            pl.BlockSpec((R, F), lambda p, j: (jnp.where(p == 1, j, 0), 0)))
    in_specs.append(pl.BlockSpec((F, F), lambda p, j: (0, 0)))

    def run(*args):
        return pl.pallas_call(
            body,
            grid=(2, NT),
            in_specs=in_specs,
            out_specs=[
                pl.BlockSpec((R, F), lambda p, j: (jnp.where(p == 1, j, 0), 0)),
                pl.BlockSpec((NQ, R, QF),
                             lambda p, j: (0, jnp.where(p == 1, j, 0), 0)),
            ],
            out_shape=[
                jax.ShapeDtypeStruct((N, F), jnp.float32),
                jax.ShapeDtypeStruct((NQ, N, QF), jnp.float32),
            ],
            scratch_shapes=[
                pltpu.VMEM((NQ, QF), jnp.float32),
                pltpu.VMEM((NQ, QF), jnp.float32),
            ],
        )(*args)

    return run


_tcmid_nores = _make_tcmid(False)
_tcmid_res = _make_tcmid(True)


def _tcfinal_body(agg_ref, dinv_ref, gb_ref, xp_ref, batch_ref,
                  wm_ref, bm_ref, out_ref, s_acc, ss_acc, sums_acc, cnt_acc):
    p = pl.program_id(0)
    j = pl.program_id(1)
    dinv = dinv_ref[...]

    @pl.when(p == 0)
    def _():
        _accum_stats(agg_ref, dinv, s_acc, ss_acc, j)

    @pl.when(p == 1)
    def _():
        @pl.when(j == 0)
        def _():
            sums_acc[...] = jnp.zeros((G, F), jnp.float32)
            cnt_acc[...] = jnp.zeros((G, 1), jnp.float32)

        x = _bn_tile(agg_ref, dinv, gb_ref[...], s_acc, ss_acc)
        x = x + xp_ref[...]
        b = batch_ref[...].reshape(1, R)
        gids = lax.broadcasted_iota(jnp.int32, (G, R), 0)
        ind = (gids == b).astype(jnp.float32)
        sums_acc[...] += jnp.dot(ind, x, preferred_element_type=jnp.float32)
        cnt_acc[...] += jnp.sum(ind, axis=1, keepdims=True)

        @pl.when(j == NT - 1)
        def _():
            hg = sums_acc[...] / jnp.maximum(cnt_acc[...], 1.0)
            out_ref[...] = (
                jnp.dot(hg, wm_ref[...], preferred_element_type=jnp.float32)
                + bm_ref[0:1, :])


def _tcfinal(agg, dinv, gb, xp, batch3, wm, bm8):
    return pl.pallas_call(
        _tcfinal_body,
        grid=(2, NT),
        in_specs=[
            pl.BlockSpec((NQ, R, QF), lambda p, j: (0, j, 0)),
            pl.BlockSpec((R, 1), lambda p, j: (j, 0)),
            pl.BlockSpec((2 * NQ, QF), lambda p, j: (0, 0)),
            pl.BlockSpec((R, F), lambda p, j: (jnp.where(p == 1, j, 0), 0)),
            pl.BlockSpec((1, 1, R),
                         lambda p, j: (jnp.where(p == 1, j, 0), 0, 0)),
            pl.BlockSpec((F, NCLS), lambda p, j: (0, 0)),
            pl.BlockSpec((8, NCLS), lambda p, j: (0, 0)),
        ],
        out_specs=pl.BlockSpec((G, NCLS), lambda p, j: (0, 0)),
        out_shape=jax.ShapeDtypeStruct((G, NCLS), jnp.float32),
        scratch_shapes=[
            pltpu.VMEM((NQ, QF), jnp.float32),
            pltpu.VMEM((NQ, QF), jnp.float32),
            pltpu.VMEM((G, F), jnp.float32),
            pltpu.VMEM((G, 1), jnp.float32),
        ],
    )(agg, dinv, gb, xp, batch3, wm, bm8)


def kernel(h, edge_index, batch, e, W0, b0, g0, be0, W1, b1, g1, be1,
           W2, b2, g2, be2, W3, b3, g3, be3, Wm, bm):
    src = edge_index[0]
    dst = edge_index[1]
    _deg_kernel, _agg_kernel = _sc_kernels()

    deg = _deg_kernel(dst).reshape(N, 1)
    y, dinv = _tc0(h, W0, deg)

    gbs = [jnp.concatenate([g.reshape(NQ, QF), be.reshape(NQ, QF)], axis=0)
           for g, be in ((g0, be0), (g1, be1), (g2, be2), (g3, be3))]
    ws = [W1, W2, W3]

    x = None
    for i in range(3):
        agg = _agg_kernel(y, src, dst)
        if i == 0:
            x, y = _tcmid_nores(agg, dinv, gbs[i], ws[i])
        else:
            x, y = _tcmid_res(agg, dinv, gbs[i], x, ws[i])

    agg = _agg_kernel(y, src, dst)
    batch3 = batch.reshape(NT, 1, R)
    bm8 = jnp.broadcast_to(bm.reshape(1, NCLS), (8, NCLS))
    return _tcfinal(agg, dinv, gbs[3], x, batch3, Wm, bm8)


# speedup vs baseline: 1.0747x; 1.0045x over previous
import functools

import jax
import jax.numpy as jnp
from jax import lax
from jax.experimental import pallas as pl
from jax.experimental.pallas import tpu as pltpu
from jax.experimental.pallas import tpu_sc as plsc

N = 10000
E = 160000
G = 128
EPS = 1e-5
NCLS = 10

QF = 128
NQ = 4
F = QF * NQ
R = 1000
NT = N // R

NCORES = 2
NSUB = 16
CH = 80
NCHT = E // CH
CPWF = (NCHT + NSUB - 1) // NSUB
NEXTRA = NCHT - (CPWF - 1) * NSUB
_DIAG_NO_SCATTER = False
_DIAG_NO_GATHER = False
NBUF = 4
GLAG = 3
IDXB = 6
RCH = 80
NRCH = N // RCH
RITER = (NRCH + NSUB - 1) // NSUB


def _deg_body(dst_hbm, deg_hbm, idx_b, ones_v, row_v, acc_sh, semi, sems):
    c = lax.axis_index("c")
    s = lax.axis_index("s")

    @pl.when(c == 0)
    def _():
        one16 = jnp.full((16,), 1.0, jnp.float32)
        for i in range(CH // 16):
            ones_v[pl.ds(i * 16, 16)] = one16
        for i in range(RCH // 16):
            row_v[pl.ds(i * 16, 16)] = one16

        def init_body(k, _):
            ch = s + k * NSUB

            @pl.when(ch < NRCH)
            def _():
                pltpu.sync_copy(row_v, acc_sh.at[pl.ds(ch * RCH, RCH)])
            return _

        lax.fori_loop(0, RITER, init_body, None)
        plsc.subcore_barrier()

        nch = jnp.where(s < NEXTRA, CPWF, CPWF - 1)

        def i_desc(k, b):
            off = (s + k * NSUB) * CH
            return pltpu.make_async_copy(dst_hbm.at[pl.ds(off, CH)],
                                         idx_b.at[b], semi.at[b])

        def s_desc(b):
            return pltpu.make_async_copy(ones_v, acc_sh.at[idx_b.at[b]],
                                         sems.at[b])

        def s_start(b):
            pltpu.async_copy(ones_v, acc_sh.at[idx_b.at[b]], sems.at[b],
                             add=True)

        i_desc(0, 0).start()
        i_desc(1, 1).start()

        def body(k, _):
            b = lax.rem(k, IDXB)

            @pl.when(k >= 3)
            def _():
                s_desc(lax.rem(k - 3, IDXB)).wait()

            @pl.when(k + 2 < nch)
            def _():
                i_desc(k + 2, lax.rem(k + 2, IDXB)).start()

            i_desc(k, b).wait()
            s_start(b)
            return _

        lax.fori_loop(0, nch, body, None)
        for d in range(3):
            @pl.when(nch - 1 - d >= 0)
            def _():
                s_desc(lax.rem(nch - 1 - d, IDXB)).wait()
        plsc.subcore_barrier()

        def out_body(k, _):
            ch = s + k * NSUB

            @pl.when(ch < NRCH)
            def _():
                pltpu.sync_copy(acc_sh.at[pl.ds(ch * RCH, RCH)], row_v)
                pltpu.sync_copy(row_v, deg_hbm.at[pl.ds(ch * RCH, RCH)])
            return _

        lax.fori_loop(0, RITER, out_body, None)


def _agg_body(y_hbm, src_hbm, dst_hbm, agg_hbm,
              isrc_b, idst_b, stage_v,
              acc_sh, semg, sems, semi):
    c = lax.axis_index("c")
    s = lax.axis_index("s")
    nch = jnp.where(s < NEXTRA, CPWF, CPWF - 1)

    nrit = jnp.where(s < NRCH - (RITER - 1) * NSUB, RITER, RITER - 1)

    def _ring_rows(ext_ref, riter, to_acc):
        def slot(b):
            return stage_v.at[b, pl.ds(0, RCH), :]

        def rows(r, ref):
            ch = s + r * NSUB
            return ref.at[pl.ds(ch * RCH, RCH), :]

        def d_in(r, b):
            if to_acc:
                return pltpu.make_async_copy(rows(r, ext_ref), slot(b),
                                             semg.at[b])
            return pltpu.make_async_copy(rows(r, acc_sh), slot(b),
                                         semg.at[b])

        def d_out(r, b):
            if to_acc:
                return pltpu.make_async_copy(slot(b), rows(r, acc_sh),
                                             sems.at[b])
            return pltpu.make_async_copy(slot(b), rows(r, ext_ref),
                                         sems.at[b])

        def body(r, _):
            b = lax.rem(r, NBUF)

            @pl.when(r >= NBUF)
            def _():
                d_out(r - NBUF, b).wait()

            d_in(r, b).start()

            @pl.when(r >= 1)
            def _():
                bp = lax.rem(r - 1, NBUF)
                d_in(r - 1, bp).wait()
                d_out(r - 1, bp).start()
            return _

        lax.fori_loop(0, riter, body, None)
        rl = riter - 1
        bl = lax.rem(rl, NBUF)
        d_in(rl, bl).wait()
        d_out(rl, bl).start()
        for d in range(NBUF):
            rr = rl - d

            @pl.when(rr >= 0)
            def _():
                d_out(rr, lax.rem(rr, NBUF)).wait()

    for qi in range(NQ // NCORES):
        q = c * (NQ // NCORES) + qi
        yq = y_hbm.at[q]

        _ring_rows(yq, nrit, to_acc=True)
        plsc.subcore_barrier()

        def i_descs(k, b):
            off = (s + k * NSUB) * CH
            return (pltpu.make_async_copy(src_hbm.at[pl.ds(off, CH)],
                                          isrc_b.at[b], semi.at[b]),
                    pltpu.make_async_copy(dst_hbm.at[pl.ds(off, CH)],
                                          idst_b.at[b], semi.at[b]))

        def i_start(k, b):
            d0, d1 = i_descs(k, b)
            d0.start()
            d1.start()

        def i_wait(k, b):
            d0, d1 = i_descs(k, b)
            d0.wait()
            d1.wait()

        def g_desc(k, b):
            return pltpu.make_async_copy(
                yq.at[isrc_b.at[lax.rem(k, IDXB)]], stage_v.at[b],
                semg.at[b])

        def s_dst(k):
            return acc_sh.at[idst_b.at[lax.rem(k, IDXB)]]

        def s_wait(k, b):
            pltpu.make_async_copy(stage_v.at[b], s_dst(k), sems.at[b]).wait()

        def s_start(k, b):
            pltpu.async_copy(stage_v.at[b], s_dst(k), sems.at[b], add=True)

        i_start(0, 0)
        i_start(1, 1)

        def edge_body(k, _):
            b = lax.rem(k, NBUF)

            if not _DIAG_NO_SCATTER:
                @pl.when(k >= NBUF)
                def _():
                    s_wait(k - NBUF, b)

            @pl.when(k + 2 < nch)
            def _():
                i_start(k + 2, lax.rem(k + 2, IDXB))

            i_wait(k, lax.rem(k, IDXB))
            if not _DIAG_NO_GATHER:
                g_desc(k, b).start()

            @pl.when(k >= GLAG)
            def _():
                bp = lax.rem(k - GLAG, NBUF)
                if not _DIAG_NO_GATHER:
                    g_desc(k - GLAG, bp).wait()
                if not _DIAG_NO_SCATTER:
                    s_start(k - GLAG, bp)
            return _

        lax.fori_loop(0, nch, edge_body, None)
        for d in range(GLAG - 1, -1, -1):
            kk = nch - 1 - d
            bk = lax.rem(kk, NBUF)
            if not _DIAG_NO_GATHER:
                g_desc(kk, bk).wait()
            if not _DIAG_NO_SCATTER:
                s_start(kk, bk)
        if not _DIAG_NO_SCATTER:
            for d in range(NBUF):
                kk = nch - 1 - d
                s_wait(kk, lax.rem(kk, NBUF))

        plsc.subcore_barrier()

        _ring_rows(agg_hbm.at[q], nrit, to_acc=False)


@functools.lru_cache(maxsize=None)
def _sc_kernels():
    mesh = plsc.VectorSubcoreMesh(
        core_axis_name="c", subcore_axis_name="s",
        num_cores=NCORES, num_subcores=NSUB)
    deg = pl.kernel(
        _deg_body,
        out_type=jax.ShapeDtypeStruct((N,), jnp.float32),
        mesh=mesh,
        scratch_types=[
            pltpu.VMEM((IDXB, CH), jnp.int32),
            pltpu.VMEM((CH,), jnp.float32),
            pltpu.VMEM((RCH,), jnp.float32),
            pltpu.VMEM_SHARED((N,), jnp.float32),
            pltpu.SemaphoreType.DMA((IDXB,)),
            pltpu.SemaphoreType.DMA((IDXB,)),
        ],
    )
    agg = pl.kernel(
        _agg_body,
        out_type=jax.ShapeDtypeStruct((NQ, N, QF), jnp.float32),
        mesh=mesh,
        compiler_params=pltpu.CompilerParams(use_tc_tiling_on_sc=False),
        scratch_types=[
            pltpu.VMEM((IDXB, CH), jnp.int32),
            pltpu.VMEM((IDXB, CH), jnp.int32),
            pltpu.VMEM((NBUF, CH, QF), jnp.float32),
            pltpu.VMEM_SHARED((N, QF), jnp.float32),
            pltpu.SemaphoreType.DMA((NBUF,)),
            pltpu.SemaphoreType.DMA((NBUF,)),
            pltpu.SemaphoreType.DMA((IDXB,)),
        ],
    )
    return deg, agg


def _tc0_body(h_ref, w_ref, deg_ref, y_ref, dinv_ref):
    dinv = lax.rsqrt(deg_ref[...])
    dinv_ref[...] = dinv
    xw = jnp.dot(h_ref[...], w_ref[...], preferred_element_type=jnp.float32)
    y = xw * dinv
    for q in range(NQ):
        y_ref[q] = y[:, q * QF:(q + 1) * QF]


def _tc0(h, w0, deg):
    return pl.pallas_call(
        _tc0_body,
        grid=(NT,),
        in_specs=[
            pl.BlockSpec((R, 256), lambda j: (j, 0)),
            pl.BlockSpec((256, F), lambda j: (0, 0)),
            pl.BlockSpec((R, 1), lambda j: (j, 0)),
        ],
        out_specs=[
            pl.BlockSpec((NQ, R, QF), lambda j: (0, j, 0)),
            pl.BlockSpec((R, 1), lambda j: (j, 0)),
        ],
        out_shape=[
            jax.ShapeDtypeStruct((NQ, N, QF), jnp.float32),
            jax.ShapeDtypeStruct((N, 1), jnp.float32),
        ],
    )(h, w0, deg)


def _accum_stats(agg_ref, dinv, s_acc, ss_acc, j):
    @pl.when(j == 0)
    def _():
        s_acc[...] = jnp.zeros((NQ, QF), jnp.float32)
        ss_acc[...] = jnp.zeros((NQ, QF), jnp.float32)

    for q in range(NQ):
        t = agg_ref[q] * dinv
        s_acc[q, :] += jnp.sum(t, axis=0)
        ss_acc[q, :] += jnp.sum(t * t, axis=0)


def _bn_tile(agg_ref, dinv, gb, s_acc, ss_acc):
    mean = s_acc[...] * (1.0 / N)
    var = ss_acc[...] * (1.0 / N) - mean * mean
    rstd = lax.rsqrt(var + EPS)
    cols = []
    for q in range(NQ):
        t = agg_ref[q] * dinv
        cols.append((t - mean[q]) * rstd[q] * gb[q] + gb[NQ + q])
    return jnp.maximum(jnp.concatenate(cols, axis=1), 0.0)


def _make_tcmid(residual):
    def body(agg_ref, dinv_ref, gb_ref, *rest):
        if residual:
            xp_ref, w_ref, x_ref, y_ref, s_acc, ss_acc = rest
        else:
            w_ref, x_ref, y_ref, s_acc, ss_acc = rest
        p = pl.program_id(0)
        j = pl.program_id(1)
        dinv = dinv_ref[...]

        @pl.when(p == 0)
        def _():
            _accum_stats(agg_ref, dinv, s_acc, ss_acc, j)

        @pl.when(p == 1)
        def _():
            x = _bn_tile(agg_ref, dinv, gb_ref[...], s_acc, ss_acc)
            if residual:
                x = x + xp_ref[...]
            x_ref[...] = x
            y = jnp.dot(x, w_ref[...],
                        preferred_element_type=jnp.float32) * dinv
            for q in range(NQ):
                y_ref[q] = y[:, q * QF:(q + 1) * QF]

    in_specs = [
        pl.BlockSpec((NQ, R, QF), lambda p, j: (0, j, 0)),
        pl.BlockSpec((R, 1), lambda p, j: (j, 0)),
        pl.BlockSpec((2 * NQ, QF), lambda p, j: (0, 0)),
    ]
    if residual:
        in_specs.append(
            pl.BlockSpec((R, F), lambda p, j: (jnp.where(p == 1, j, 0), 0)))
    in_specs.append(pl.BlockSpec((F, F), lambda p, j: (0, 0)))

    def run(*args):
        return pl.pallas_call(
            body,
            grid=(2, NT),
            in_specs=in_specs,
            out_specs=[
                pl.BlockSpec((R, F), lambda p, j: (jnp.where(p == 1, j, 0), 0)),
                pl.BlockSpec((NQ, R, QF),
                             lambda p, j: (0, jnp.where(p == 1, j, 0), 0)),
            ],
            out_shape=[
                jax.ShapeDtypeStruct((N, F), jnp.float32),
                jax.ShapeDtypeStruct((NQ, N, QF), jnp.float32),
            ],
            scratch_shapes=[
                pltpu.VMEM((NQ, QF), jnp.float32),
                pltpu.VMEM((NQ, QF), jnp.float32),
            ],
        )(*args)

    return run


_tcmid_nores = _make_tcmid(False)
_tcmid_res = _make_tcmid(True)


def _tcfinal_body(agg_ref, dinv_ref, gb_ref, xp_ref, batch_ref,
                  wm_ref, bm_ref, out_ref, s_acc, ss_acc, sums_acc, cnt_acc):
    p = pl.program_id(0)
    j = pl.program_id(1)
    dinv = dinv_ref[...]

    @pl.when(p == 0)
    def _():
        _accum_stats(agg_ref, dinv, s_acc, ss_acc, j)

    @pl.when(p == 1)
    def _():
        @pl.when(j == 0)
        def _():
            sums_acc[...] = jnp.zeros((G, F), jnp.float32)
            cnt_acc[...] = jnp.zeros((G, 1), jnp.float32)

        x = _bn_tile(agg_ref, dinv, gb_ref[...], s_acc, ss_acc)
        x = x + xp_ref[...]
        b = batch_ref[...].reshape(1, R)
        gids = lax.broadcasted_iota(jnp.int32, (G, R), 0)
        ind = (gids == b).astype(jnp.float32)
        sums_acc[...] += jnp.dot(ind, x, preferred_element_type=jnp.float32)
        cnt_acc[...] += jnp.sum(ind, axis=1, keepdims=True)

        @pl.when(j == NT - 1)
        def _():
            hg = sums_acc[...] / jnp.maximum(cnt_acc[...], 1.0)
            out_ref[...] = (
                jnp.dot(hg, wm_ref[...], preferred_element_type=jnp.float32)
                + bm_ref[0:1, :])


def _tcfinal(agg, dinv, gb, xp, batch3, wm, bm8):
    return pl.pallas_call(
        _tcfinal_body,
        grid=(2, NT),
        in_specs=[
            pl.BlockSpec((NQ, R, QF), lambda p, j: (0, j, 0)),
            pl.BlockSpec((R, 1), lambda p, j: (j, 0)),
            pl.BlockSpec((2 * NQ, QF), lambda p, j: (0, 0)),
            pl.BlockSpec((R, F), lambda p, j: (jnp.where(p == 1, j, 0), 0)),
            pl.BlockSpec((1, 1, R),
                         lambda p, j: (jnp.where(p == 1, j, 0), 0, 0)),
            pl.BlockSpec((F, NCLS), lambda p, j: (0, 0)),
            pl.BlockSpec((8, NCLS), lambda p, j: (0, 0)),
        ],
        out_specs=pl.BlockSpec((G, NCLS), lambda p, j: (0, 0)),
        out_shape=jax.ShapeDtypeStruct((G, NCLS), jnp.float32),
        scratch_shapes=[
            pltpu.VMEM((NQ, QF), jnp.float32),
            pltpu.VMEM((NQ, QF), jnp.float32),
            pltpu.VMEM((G, F), jnp.float32),
            pltpu.VMEM((G, 1), jnp.float32),
        ],
    )(agg, dinv, gb, xp, batch3, wm, bm8)


def kernel(h, edge_index, batch, e, W0, b0, g0, be0, W1, b1, g1, be1,
           W2, b2, g2, be2, W3, b3, g3, be3, Wm, bm):
    src = edge_index[0]
    dst = edge_index[1]
    _deg_kernel, _agg_kernel = _sc_kernels()

    deg = _deg_kernel(dst).reshape(N, 1)
    y, dinv = _tc0(h, W0, deg)

    gbs = [jnp.concatenate([g.reshape(NQ, QF), be.reshape(NQ, QF)], axis=0)
           for g, be in ((g0, be0), (g1, be1), (g2, be2), (g3, be3))]
    ws = [W1, W2, W3]

    x = None
    for i in range(3):
        agg = _agg_kernel(y, src, dst)
        if i == 0:
            x, y = _tcmid_nores(agg, dinv, gbs[i], ws[i])
        else:
            x, y = _tcmid_res(agg, dinv, gbs[i], x, ws[i])

    agg = _agg_kernel(y, src, dst)
    batch3 = batch.reshape(NT, 1, R)
    bm8 = jnp.broadcast_to(bm.reshape(1, NCLS), (8, NCLS))
    return _tcfinal(agg, dinv, gbs[3], x, batch3, Wm, bm8)
